# Initial kernel scaffold; baseline (speedup 1.0000x reference)
import jax
import jax.numpy as jnp
from jax.experimental import pallas as pl


def kernel(x, edge_index, W1, b1, W2, b2):
    raise NotImplementedError("write your pallas kernel here")


# speedup vs baseline: 10.9907x; 10.9907x over previous
import functools

import jax
import jax.numpy as jnp
from jax import lax
from jax.experimental import pallas as pl
from jax.experimental.pallas import tpu as pltpu
from jax.experimental.pallas import tpu_sc as plsc

N = 10000
D = 128
E = 320000

NC = 2
NS = 16
NW = NC * NS
L = 16

CH = 128
CHUNKS = -(-E // (NW * CH))
EPW = CHUNKS * CH
E_PAD = EPW * NW

N_PAD = 10240
RPS = N_PAD // NS
ZR = 64
CW = 16

BLK = 2048

_MESH = plsc.VectorSubcoreMesh(core_axis_name="c", subcore_axis_name="s")


@functools.partial(
    pl.kernel,
    out_type=jax.ShapeDtypeStruct((NC, N_PAD, CW), jnp.float32),
    mesh=_MESH,
    scratch_types=[
        pltpu.VMEM((CH, CW), jnp.float32),
        pltpu.VMEM((ZR, CW), jnp.float32),
        pltpu.VMEM((CH,), jnp.int32),
        pltpu.VMEM_SHARED((N_PAD, CW), jnp.float32),
    ],
)
def _deg_kernel(dst_hbm, out_hbm, ones_v, zer_v, didx, acc):
    cid = lax.axis_index("c")
    sid = lax.axis_index("s")
    wid = sid * NC + cid

    one16 = jnp.ones((L,), jnp.float32)
    zero16 = jnp.zeros((L,), jnp.float32)

    def fill(i, c):
        ones_v[i, :] = one16
        return c

    lax.fori_loop(0, CH, fill, 0)

    def zfill(i, c):
        zer_v[i, :] = zero16
        return c

    lax.fori_loop(0, ZR, zfill, 0)

    r0 = sid * RPS

    def zblk(t, c):
        pltpu.sync_copy(zer_v, acc.at[pl.ds(r0 + t * ZR, ZR)])
        return c

    lax.fori_loop(0, RPS // ZR, zblk, 0)
    plsc.subcore_barrier()

    def chunk(i, c):
        pltpu.sync_copy(dst_hbm.at[wid, i], didx)
        pltpu.sync_copy(ones_v, acc.at[didx], add=True)
        return c

    lax.fori_loop(0, CHUNKS, chunk, 0)
    plsc.subcore_barrier()

    pltpu.sync_copy(acc.at[pl.ds(r0, RPS)], out_hbm.at[cid, pl.ds(r0, RPS)])


@functools.partial(
    pl.kernel,
    out_type=jax.ShapeDtypeStruct((NC, N_PAD, D), jnp.float32),
    mesh=_MESH,
    scratch_types=[
        pltpu.VMEM((CH,), jnp.int32),
        pltpu.VMEM((CH,), jnp.int32),
        pltpu.VMEM((CH, D), jnp.float32),
        pltpu.VMEM((ZR, D), jnp.float32),
        pltpu.VMEM_SHARED((N_PAD, D), jnp.float32),
        pltpu.SemaphoreType.DMA,
    ],
)
def _edge_kernel(y_hbm, src_hbm, dst_hbm, out_hbm, sidx, didx, rows, zer_v,
                 acc, sem):
    cid = lax.axis_index("c")
    sid = lax.axis_index("s")
    wid = sid * NC + cid

    zero16 = jnp.zeros((L,), jnp.float32)

    def zfill(i, c):
        for j in range(D // L):
            zer_v[i, pl.ds(j * L, L)] = zero16
        return c

    lax.fori_loop(0, ZR, zfill, 0)

    r0 = sid * RPS

    def zblk(t, c):
        pltpu.sync_copy(zer_v, acc.at[pl.ds(r0 + t * ZR, ZR)])
        return c

    lax.fori_loop(0, RPS // ZR, zblk, 0)
    plsc.subcore_barrier()

    def chunk(i, c):
        pltpu.sync_copy(src_hbm.at[wid, i], sidx)
        pltpu.sync_copy(dst_hbm.at[wid, i], didx)
        pltpu.async_copy(y_hbm.at[sidx], rows, sem).wait()
        pltpu.sync_copy(rows, acc.at[didx], add=True)
        return c

    lax.fori_loop(0, CHUNKS, chunk, 0)
    plsc.subcore_barrier()

    pltpu.sync_copy(acc.at[pl.ds(r0, RPS)], out_hbm.at[cid, pl.ds(r0, RPS)])


def _tc_pre_body(counts_ref, x_ref, w_ref, y_ref, dis_ref):
    c = counts_ref[...]
    deg = c[0, :, 0] + c[1, :, 0] + 1.0
    dis = lax.rsqrt(deg)[:, None]
    xw = jnp.dot(x_ref[...], w_ref[...], preferred_element_type=jnp.float32)
    y_ref[...] = xw * dis
    dis_ref[...] = dis


def _tc_pre(counts, x_p, w):
    grid = (N_PAD // BLK,)
    return pl.pallas_call(
        _tc_pre_body,
        grid=grid,
        in_specs=[
            pl.BlockSpec((NC, BLK, CW), lambda i: (0, i, 0)),
            pl.BlockSpec((BLK, D), lambda i: (i, 0)),
            pl.BlockSpec((D, D), lambda i: (0, 0)),
        ],
        out_specs=[
            pl.BlockSpec((BLK, D), lambda i: (i, 0)),
            pl.BlockSpec((BLK, 1), lambda i: (i, 0)),
        ],
        out_shape=[
            jax.ShapeDtypeStruct((N_PAD, D), jnp.float32),
            jax.ShapeDtypeStruct((N_PAD, 1), jnp.float32),
        ],
    )(counts, x_p, w)


def _tc_mid_body(a_ref, y_ref, dis_ref, b_ref, w_ref, y2_ref):
    dis = dis_ref[...]
    t = jnp.tanh(dis * (a_ref[0] + a_ref[1] + y_ref[...]) + b_ref[...])
    y2_ref[...] = jnp.dot(
        t, w_ref[...], preferred_element_type=jnp.float32) * dis


def _tc_mid(a, y, dis, b, w):
    grid = (N_PAD // BLK,)
    return pl.pallas_call(
        _tc_mid_body,
        grid=grid,
        in_specs=[
            pl.BlockSpec((NC, BLK, D), lambda i: (0, i, 0)),
            pl.BlockSpec((BLK, D), lambda i: (i, 0)),
            pl.BlockSpec((BLK, 1), lambda i: (i, 0)),
            pl.BlockSpec((1, D), lambda i: (0, 0)),
            pl.BlockSpec((D, D), lambda i: (0, 0)),
        ],
        out_specs=pl.BlockSpec((BLK, D), lambda i: (i, 0)),
        out_shape=jax.ShapeDtypeStruct((N_PAD, D), jnp.float32),
    )(a, y, dis, b, w)


def _tc_post_body(a_ref, y_ref, dis_ref, b_ref, h_ref):
    h_ref[...] = jnp.tanh(
        dis_ref[...] * (a_ref[0] + a_ref[1] + y_ref[...]) + b_ref[...])


def _tc_post(a, y, dis, b):
    grid = (N_PAD // BLK,)
    return pl.pallas_call(
        _tc_post_body,
        grid=grid,
        in_specs=[
            pl.BlockSpec((NC, BLK, D), lambda i: (0, i, 0)),
            pl.BlockSpec((BLK, D), lambda i: (i, 0)),
            pl.BlockSpec((BLK, 1), lambda i: (i, 0)),
            pl.BlockSpec((1, D), lambda i: (0, 0)),
        ],
        out_specs=pl.BlockSpec((BLK, D), lambda i: (i, 0)),
        out_shape=jax.ShapeDtypeStruct((N_PAD, D), jnp.float32),
    )(a, y, dis, b)


def kernel(x, edge_index, W1, b1, W2, b2):
    n, d = x.shape
    e = edge_index.shape[1]
    src = edge_index[0].astype(jnp.int32)
    dst = edge_index[1].astype(jnp.int32)
    fill = jnp.full((E_PAD - e,), n, jnp.int32)
    src_w = jnp.concatenate([src, fill]).reshape(NW, CHUNKS, CH)
    dst_w = jnp.concatenate([dst, fill]).reshape(NW, CHUNKS, CH)
    x_p = jnp.pad(x, ((0, N_PAD - n), (0, 0)))

    counts = _deg_kernel(dst_w)
    y1, dis = _tc_pre(counts, x_p, W1)
    a1 = _edge_kernel(y1, src_w, dst_w)
    y2 = _tc_mid(a1, y1, dis, b1.reshape(1, d), W2)
    a2 = _edge_kernel(y2, src_w, dst_w)
    h = _tc_post(a2, y2, dis, b2.reshape(1, d))
    return (h[:n], x)


# speedup vs baseline: 14.7276x; 1.3400x over previous
import functools

import jax
import jax.numpy as jnp
from jax import lax
from jax.experimental import pallas as pl
from jax.experimental.pallas import tpu as pltpu
from jax.experimental.pallas import tpu_sc as plsc

N = 10000
D = 128
E = 320000

NC = 2
NS = 16
NW = NC * NS
L = 16

CH = 128
CHUNKS = -(-E // (NW * CH))
EPW = CHUNKS * CH
E_PAD = EPW * NW

N_PAD = 10240
RPS = N_PAD // NS
ZR = 64
CW = 16

BLK = 2048

_MESH = plsc.VectorSubcoreMesh(core_axis_name="c", subcore_axis_name="s")


@functools.partial(
    pl.kernel,
    out_type=jax.ShapeDtypeStruct((NC, N_PAD, CW), jnp.float32),
    mesh=_MESH,
    scratch_types=[
        pltpu.VMEM((CH, CW), jnp.float32),
        pltpu.VMEM((ZR, CW), jnp.float32),
        pltpu.VMEM((CH,), jnp.int32),
        pltpu.VMEM_SHARED((N_PAD, CW), jnp.float32),
    ],
)
def _deg_kernel(dst_hbm, out_hbm, ones_v, zer_v, didx, acc):
    cid = lax.axis_index("c")
    sid = lax.axis_index("s")
    wid = sid * NC + cid

    one16 = jnp.ones((L,), jnp.float32)
    zero16 = jnp.zeros((L,), jnp.float32)

    def fill(i, c):
        ones_v[i, :] = one16
        return c

    lax.fori_loop(0, CH, fill, 0)

    def zfill(i, c):
        zer_v[i, :] = zero16
        return c

    lax.fori_loop(0, ZR, zfill, 0)

    r0 = sid * RPS

    def zblk(t, c):
        pltpu.sync_copy(zer_v, acc.at[pl.ds(r0 + t * ZR, ZR)])
        return c

    lax.fori_loop(0, RPS // ZR, zblk, 0)
    plsc.subcore_barrier()

    def chunk(i, c):
        pltpu.sync_copy(dst_hbm.at[pl.ds(wid * EPW + i * CH, CH)], didx)
        pltpu.sync_copy(ones_v, acc.at[didx], add=True)
        return c

    lax.fori_loop(0, CHUNKS, chunk, 0)
    plsc.subcore_barrier()

    pltpu.sync_copy(acc.at[pl.ds(r0, RPS)], out_hbm.at[cid, pl.ds(r0, RPS)])


@functools.partial(
    pl.kernel,
    out_type=jax.ShapeDtypeStruct((NC, N_PAD, D), jnp.float32),
    mesh=_MESH,
    scratch_types=[
        pltpu.VMEM((CH,), jnp.int32),
        pltpu.VMEM((CH,), jnp.int32),
        pltpu.VMEM((CH,), jnp.int32),
        pltpu.VMEM((CH,), jnp.int32),
        pltpu.VMEM((CH, D), jnp.float32),
        pltpu.VMEM((CH, D), jnp.float32),
        pltpu.VMEM_SHARED((N_PAD, D), jnp.float32),
        pltpu.SemaphoreType.DMA,
        pltpu.SemaphoreType.DMA,
        pltpu.SemaphoreType.DMA,
        pltpu.SemaphoreType.DMA,
        pltpu.SemaphoreType.DMA,
    ],
)
def _edge_kernel(y_hbm, src_hbm, dst_hbm, out_hbm, sidx0, sidx1, didx0,
                 didx1, rows0, rows1, acc, semg, semi, semd, sems0, sems1):
    cid = lax.axis_index("c")
    sid = lax.axis_index("s")
    wid = sid * NC + cid

    rows = (rows0, rows1)
    sring = (sidx0, sidx1)
    dring = (didx0, didx1)
    sems = (sems0, sems1)

    def sld(k, b):
        pltpu.async_copy(src_hbm.at[pl.ds(wid * EPW + k * CH, CH)], sring[b], semi)

    def wait_sld(k, b):
        pltpu.make_async_copy(src_hbm.at[pl.ds(wid * EPW + k * CH, CH)], sring[b], semi).wait()

    def dld(k, b):
        pltpu.async_copy(dst_hbm.at[pl.ds(wid * EPW + k * CH, CH)], dring[b], semd)

    def wait_dld(k, b):
        pltpu.make_async_copy(dst_hbm.at[pl.ds(wid * EPW + k * CH, CH)], dring[b], semd).wait()

    def gath(k, b):
        pltpu.async_copy(y_hbm.at[sring[b]], rows[b], semg)

    def wait_gath(k, b):
        pltpu.make_async_copy(y_hbm.at[sring[b]], rows[b], semg).wait()

    def scat(k, b):
        pltpu.async_copy(rows[b], acc.at[dring[b]], sems[b], add=True)

    def wait_scat(k, b):
        pltpu.make_async_copy(rows[b], acc.at[dring[b]], sems[b]).wait()

    sld(0, 0)
    sld(1, 1)
    dld(0, 0)
    dld(1, 1)

    zero16 = jnp.zeros((L,), jnp.float32)

    def zfill(i, c):
        for j in range(D // L):
            rows0[i, pl.ds(j * L, L)] = zero16
        return c

    lax.fori_loop(0, CH, zfill, 0)

    r0 = sid * RPS

    def zblk(t, c):
        pltpu.sync_copy(rows0, acc.at[pl.ds(r0 + t * CH, CH)])
        return c

    lax.fori_loop(0, RPS // CH, zblk, 0)
    plsc.subcore_barrier()

    wait_sld(0, 0)
    gath(0, 0)
    wait_gath(0, 0)
    sld(2, 0)
    wait_sld(1, 1)
    gath(1, 1)
    wait_dld(0, 0)
    scat(0, 0)

    def step(k, b):
        bo = 1 - b
        wait_gath(k, b)
        sld(k + 2, b)
        wait_scat(k - 1, bo)
        dld(k + 1, bo)
        wait_sld(k + 1, bo)
        gath(k + 1, bo)
        wait_dld(k, b)
        scat(k, b)

    def pair(p, c):
        step(2 * p + 1, 1)
        step(2 * p + 2, 0)
        return c

    lax.fori_loop(0, (CHUNKS - 3) // 2, pair, 0)

    k1 = CHUNKS - 2
    wait_gath(k1, 1)
    wait_scat(k1 - 1, 0)
    dld(k1 + 1, 0)
    wait_sld(k1 + 1, 0)
    gath(k1 + 1, 0)
    wait_dld(k1, 1)
    scat(k1, 1)
    k2 = CHUNKS - 1
    wait_gath(k2, 0)
    wait_scat(k2 - 1, 1)
    wait_dld(k2, 0)
    scat(k2, 0)
    wait_scat(k2, 0)

    plsc.subcore_barrier()

    pltpu.sync_copy(acc.at[pl.ds(r0, RPS)], out_hbm.at[cid, pl.ds(r0, RPS)])


def _tc_pre_body(counts_ref, x_ref, w_ref, y_ref, dis_ref):
    c = counts_ref[...]
    deg = c[0, :, 0] + c[1, :, 0] + 1.0
    dis = lax.rsqrt(deg)[:, None]
    xw = jnp.dot(x_ref[...], w_ref[...], preferred_element_type=jnp.float32)
    y_ref[...] = xw * dis
    dis_ref[...] = dis


def _tc_pre(counts, x_p, w):
    grid = (N_PAD // BLK,)
    return pl.pallas_call(
        _tc_pre_body,
        grid=grid,
        in_specs=[
            pl.BlockSpec((NC, BLK, CW), lambda i: (0, i, 0)),
            pl.BlockSpec((BLK, D), lambda i: (i, 0)),
            pl.BlockSpec((D, D), lambda i: (0, 0)),
        ],
        out_specs=[
            pl.BlockSpec((BLK, D), lambda i: (i, 0)),
            pl.BlockSpec((BLK, 1), lambda i: (i, 0)),
        ],
        out_shape=[
            jax.ShapeDtypeStruct((N_PAD, D), jnp.float32),
            jax.ShapeDtypeStruct((N_PAD, 1), jnp.float32),
        ],
    )(counts, x_p, w)


def _tc_mid_body(a_ref, y_ref, dis_ref, b_ref, w_ref, y2_ref):
    dis = dis_ref[...]
    t = jnp.tanh(dis * (a_ref[0] + a_ref[1] + y_ref[...]) + b_ref[...])
    y2_ref[...] = jnp.dot(
        t, w_ref[...], preferred_element_type=jnp.float32) * dis


def _tc_mid(a, y, dis, b, w):
    grid = (N_PAD // BLK,)
    return pl.pallas_call(
        _tc_mid_body,
        grid=grid,
        in_specs=[
            pl.BlockSpec((NC, BLK, D), lambda i: (0, i, 0)),
            pl.BlockSpec((BLK, D), lambda i: (i, 0)),
            pl.BlockSpec((BLK, 1), lambda i: (i, 0)),
            pl.BlockSpec((1, D), lambda i: (0, 0)),
            pl.BlockSpec((D, D), lambda i: (0, 0)),
        ],
        out_specs=pl.BlockSpec((BLK, D), lambda i: (i, 0)),
        out_shape=jax.ShapeDtypeStruct((N_PAD, D), jnp.float32),
    )(a, y, dis, b, w)


def _tc_post_body(a_ref, y_ref, dis_ref, b_ref, h_ref):
    h_ref[...] = jnp.tanh(
        dis_ref[...] * (a_ref[0] + a_ref[1] + y_ref[...]) + b_ref[...])


def _tc_post(a, y, dis, b):
    grid = (N_PAD // BLK,)
    return pl.pallas_call(
        _tc_post_body,
        grid=grid,
        in_specs=[
            pl.BlockSpec((NC, BLK, D), lambda i: (0, i, 0)),
            pl.BlockSpec((BLK, D), lambda i: (i, 0)),
            pl.BlockSpec((BLK, 1), lambda i: (i, 0)),
            pl.BlockSpec((1, D), lambda i: (0, 0)),
        ],
        out_specs=pl.BlockSpec((BLK, D), lambda i: (i, 0)),
        out_shape=jax.ShapeDtypeStruct((N_PAD, D), jnp.float32),
    )(a, y, dis, b)


def kernel(x, edge_index, W1, b1, W2, b2):
    n, d = x.shape
    e = edge_index.shape[1]
    src = edge_index[0].astype(jnp.int32)
    dst = edge_index[1].astype(jnp.int32)
    fill = jnp.full((E_PAD - e,), n, jnp.int32)
    src_w = jnp.concatenate([src, fill])
    dst_w = jnp.concatenate([dst, fill])
    x_p = jnp.pad(x, ((0, N_PAD - n), (0, 0)))

    counts = _deg_kernel(dst_w)
    y1, dis = _tc_pre(counts, x_p, W1)
    a1 = _edge_kernel(y1, src_w, dst_w)
    y2 = _tc_mid(a1, y1, dis, b1.reshape(1, d), W2)
    a2 = _edge_kernel(y2, src_w, dst_w)
    h = _tc_post(a2, y2, dis, b2.reshape(1, d))
    return (h[:n], x)


# speedup vs baseline: 26.2427x; 1.7819x over previous
import functools

import jax
import jax.numpy as jnp
from jax import lax
from jax.experimental import pallas as pl
from jax.experimental.pallas import tpu as pltpu
from jax.experimental.pallas import tpu_sc as plsc

N = 10000
D = 128
E = 320000

NC = 2
NS = 16
NW = NC * NS
L = 16

CH = 128
CHUNKS = -(-E // (NW * CH))
EPW = CHUNKS * CH
E_PAD = EPW * NW

N_PAD = 10240
RPS = N_PAD // NS
CW = 16

BLK = 2048

_MESH = plsc.VectorSubcoreMesh(core_axis_name="c", subcore_axis_name="s")


@functools.partial(
    pl.kernel,
    out_type=jax.ShapeDtypeStruct((NC, N_PAD, CW), jnp.float32),
    mesh=_MESH,
    scratch_types=[
        pltpu.VMEM((CH, CW), jnp.float32),
        pltpu.VMEM((CH,), jnp.int32),
        pltpu.VMEM((CH,), jnp.int32),
        pltpu.VMEM_SHARED((N_PAD, CW), jnp.float32),
        pltpu.SemaphoreType.DMA,
        pltpu.SemaphoreType.DMA,
        pltpu.SemaphoreType.DMA,
    ],
)
def _deg_kernel(dst_hbm, out_hbm, ones_v, didx0, didx1, acc, semd, sems0,
                sems1):
    cid = lax.axis_index("c")
    sid = lax.axis_index("s")
    wid = sid * NC + cid

    dring = (didx0, didx1)
    sems = (sems0, sems1)

    def dld(k, b):
        pltpu.async_copy(dst_hbm.at[pl.ds(wid * EPW + k * CH, CH)],
                         dring[b], semd)

    def wait_dld(k, b):
        pltpu.make_async_copy(dst_hbm.at[pl.ds(wid * EPW + k * CH, CH)],
                              dring[b], semd).wait()

    def scat(k, b):
        pltpu.async_copy(ones_v, acc.at[dring[b]], sems[b], add=True)

    def wait_scat(k, b):
        pltpu.make_async_copy(ones_v, acc.at[dring[b]], sems[b]).wait()

    dld(0, 0)
    dld(1, 1)

    one16 = jnp.ones((L,), jnp.float32)
    zero16 = jnp.zeros((L,), jnp.float32)

    def zfill(i, c):
        ones_v[i, :] = zero16
        return c

    lax.fori_loop(0, CH, zfill, 0)

    r0 = sid * RPS

    def zblk(t, c):
        pltpu.sync_copy(ones_v, acc.at[pl.ds(r0 + t * CH, CH)])
        return c

    lax.fori_loop(0, RPS // CH, zblk, 0)

    def fill(i, c):
        ones_v[i, :] = one16
        return c

    lax.fori_loop(0, CH, fill, 0)
    plsc.subcore_barrier()

    wait_dld(0, 0)
    scat(0, 0)

    def step(k, b):
        bo = 1 - b
        wait_scat(k - 1, bo)
        dld(k + 1, bo)
        wait_dld(k, b)
        scat(k, b)

    def pair(p, c):
        step(2 * p + 1, 1)
        step(2 * p + 2, 0)
        return c

    lax.fori_loop(0, (CHUNKS - 3) // 2, pair, 0)

    k1 = CHUNKS - 2
    wait_scat(k1 - 1, 0)
    dld(k1 + 1, 0)
    wait_dld(k1, 1)
    scat(k1, 1)
    k2 = CHUNKS - 1
    wait_scat(k2 - 1, 1)
    wait_dld(k2, 0)
    scat(k2, 0)
    wait_scat(k2, 0)

    plsc.subcore_barrier()

    pltpu.sync_copy(acc.at[pl.ds(r0, RPS)], out_hbm.at[cid, pl.ds(r0, RPS)])


@functools.partial(
    pl.kernel,
    out_type=jax.ShapeDtypeStruct((NC, N_PAD, D), jnp.float32),
    mesh=_MESH,
    scratch_types=[
        pltpu.VMEM((CH,), jnp.int32),
        pltpu.VMEM((CH,), jnp.int32),
        pltpu.VMEM((CH,), jnp.int32),
        pltpu.VMEM((CH,), jnp.int32),
        pltpu.VMEM((CH, D), jnp.float32),
        pltpu.VMEM((CH, D), jnp.float32),
        pltpu.VMEM_SHARED((N_PAD, D), jnp.float32),
        pltpu.SemaphoreType.DMA,
        pltpu.SemaphoreType.DMA,
        pltpu.SemaphoreType.DMA,
        pltpu.SemaphoreType.DMA,
        pltpu.SemaphoreType.DMA,
    ],
)
def _edge_kernel(y_hbm, src_hbm, dst_hbm, out_hbm, sidx0, sidx1, didx0,
                 didx1, rows0, rows1, acc, semg, semi, semd, sems0, sems1):
    cid = lax.axis_index("c")
    sid = lax.axis_index("s")
    wid = sid * NC + cid

    rows = (rows0, rows1)
    sring = (sidx0, sidx1)
    dring = (didx0, didx1)
    sems = (sems0, sems1)

    def sld(k, b):
        pltpu.async_copy(src_hbm.at[pl.ds(wid * EPW + k * CH, CH)], sring[b], semi)

    def wait_sld(k, b):
        pltpu.make_async_copy(src_hbm.at[pl.ds(wid * EPW + k * CH, CH)], sring[b], semi).wait()

    def dld(k, b):
        pltpu.async_copy(dst_hbm.at[pl.ds(wid * EPW + k * CH, CH)], dring[b], semd)

    def wait_dld(k, b):
        pltpu.make_async_copy(dst_hbm.at[pl.ds(wid * EPW + k * CH, CH)], dring[b], semd).wait()

    def gath(k, b):
        pltpu.async_copy(y_hbm.at[sring[b]], rows[b], semg)

    def wait_gath(k, b):
        pltpu.make_async_copy(y_hbm.at[sring[b]], rows[b], semg).wait()

    def scat(k, b):
        pltpu.async_copy(rows[b], acc.at[dring[b]], sems[b], add=True)

    def wait_scat(k, b):
        pltpu.make_async_copy(rows[b], acc.at[dring[b]], sems[b]).wait()

    sld(0, 0)
    sld(1, 1)
    dld(0, 0)
    dld(1, 1)

    zero16 = jnp.zeros((L,), jnp.float32)

    def zfill(i, c):
        for j in range(D // L):
            rows0[i, pl.ds(j * L, L)] = zero16
        return c

    lax.fori_loop(0, CH, zfill, 0)

    r0 = sid * RPS

    def zblk(t, c):
        pltpu.sync_copy(rows0, acc.at[pl.ds(r0 + t * CH, CH)])
        return c

    lax.fori_loop(0, RPS // CH, zblk, 0)
    plsc.subcore_barrier()

    wait_sld(0, 0)
    gath(0, 0)
    wait_gath(0, 0)
    sld(2, 0)
    wait_sld(1, 1)
    gath(1, 1)
    wait_dld(0, 0)
    scat(0, 0)

    def step(k, b):
        bo = 1 - b
        wait_gath(k, b)
        sld(k + 2, b)
        wait_scat(k - 1, bo)
        dld(k + 1, bo)
        wait_sld(k + 1, bo)
        gath(k + 1, bo)
        wait_dld(k, b)
        scat(k, b)

    def pair(p, c):
        step(2 * p + 1, 1)
        step(2 * p + 2, 0)
        return c

    lax.fori_loop(0, (CHUNKS - 3) // 2, pair, 0)

    k1 = CHUNKS - 2
    wait_gath(k1, 1)
    wait_scat(k1 - 1, 0)
    dld(k1 + 1, 0)
    wait_sld(k1 + 1, 0)
    gath(k1 + 1, 0)
    wait_dld(k1, 1)
    scat(k1, 1)
    k2 = CHUNKS - 1
    wait_gath(k2, 0)
    wait_scat(k2 - 1, 1)
    wait_dld(k2, 0)
    scat(k2, 0)
    wait_scat(k2, 0)

    plsc.subcore_barrier()

    pltpu.sync_copy(acc.at[pl.ds(r0, RPS)], out_hbm.at[cid, pl.ds(r0, RPS)])


def _tc_pre_body(counts_ref, x_ref, w_ref, y_ref, dis_ref):
    c = counts_ref[...]
    deg = c[0, :, 0] + c[1, :, 0] + 1.0
    dis = lax.rsqrt(deg)[:, None]
    xw = jnp.dot(x_ref[...], w_ref[...], preferred_element_type=jnp.float32)
    y_ref[...] = xw * dis
    dis_ref[...] = dis


def _tc_pre(counts, x_p, w):
    grid = (N_PAD // BLK,)
    return pl.pallas_call(
        _tc_pre_body,
        grid=grid,
        in_specs=[
            pl.BlockSpec((NC, BLK, CW), lambda i: (0, i, 0)),
            pl.BlockSpec((BLK, D), lambda i: (i, 0)),
            pl.BlockSpec((D, D), lambda i: (0, 0)),
        ],
        out_specs=[
            pl.BlockSpec((BLK, D), lambda i: (i, 0)),
            pl.BlockSpec((BLK, 1), lambda i: (i, 0)),
        ],
        out_shape=[
            jax.ShapeDtypeStruct((N_PAD, D), jnp.float32),
            jax.ShapeDtypeStruct((N_PAD, 1), jnp.float32),
        ],
    )(counts, x_p, w)


def _tc_mid_body(a_ref, y_ref, dis_ref, b_ref, w_ref, y2_ref):
    dis = dis_ref[...]
    t = jnp.tanh(dis * (a_ref[0] + a_ref[1] + y_ref[...]) + b_ref[...])
    y2_ref[...] = jnp.dot(
        t, w_ref[...], preferred_element_type=jnp.float32) * dis


def _tc_mid(a, y, dis, b, w):
    grid = (N_PAD // BLK,)
    return pl.pallas_call(
        _tc_mid_body,
        grid=grid,
        in_specs=[
            pl.BlockSpec((NC, BLK, D), lambda i: (0, i, 0)),
            pl.BlockSpec((BLK, D), lambda i: (i, 0)),
            pl.BlockSpec((BLK, 1), lambda i: (i, 0)),
            pl.BlockSpec((1, D), lambda i: (0, 0)),
            pl.BlockSpec((D, D), lambda i: (0, 0)),
        ],
        out_specs=pl.BlockSpec((BLK, D), lambda i: (i, 0)),
        out_shape=jax.ShapeDtypeStruct((N_PAD, D), jnp.float32),
    )(a, y, dis, b, w)


def _tc_post_body(a_ref, y_ref, dis_ref, b_ref, h_ref):
    h_ref[...] = jnp.tanh(
        dis_ref[...] * (a_ref[0] + a_ref[1] + y_ref[...]) + b_ref[...])


def _tc_post(a, y, dis, b):
    grid = (N_PAD // BLK,)
    return pl.pallas_call(
        _tc_post_body,
        grid=grid,
        in_specs=[
            pl.BlockSpec((NC, BLK, D), lambda i: (0, i, 0)),
            pl.BlockSpec((BLK, D), lambda i: (i, 0)),
            pl.BlockSpec((BLK, 1), lambda i: (i, 0)),
            pl.BlockSpec((1, D), lambda i: (0, 0)),
        ],
        out_specs=pl.BlockSpec((BLK, D), lambda i: (i, 0)),
        out_shape=jax.ShapeDtypeStruct((N_PAD, D), jnp.float32),
    )(a, y, dis, b)


def kernel(x, edge_index, W1, b1, W2, b2):
    n, d = x.shape
    e = edge_index.shape[1]
    src = edge_index[0].astype(jnp.int32)
    dst = edge_index[1].astype(jnp.int32)
    fill = (jnp.arange(E_PAD - e, dtype=jnp.int32) % 64) + n + 1
    src_w = jnp.concatenate([src, fill])
    dst_w = jnp.concatenate([dst, fill])
    x_p = jnp.pad(x, ((0, N_PAD - n), (0, 0)))

    counts = _deg_kernel(dst_w)
    y1, dis = _tc_pre(counts, x_p, W1)
    a1 = _edge_kernel(y1, src_w, dst_w)
    y2 = _tc_mid(a1, y1, dis, b1.reshape(1, d), W2)
    a2 = _edge_kernel(y2, src_w, dst_w)
    h = _tc_post(a2, y2, dis, b2.reshape(1, d))
    return (h[:n], x)
